# baseline (speedup 1.0000x reference)
import functools

import jax
import jax.numpy as jnp
from jax import lax
from jax.experimental import pallas as pl
from jax.experimental.pallas import tpu as pltpu
from jax.experimental.pallas import tpu_sc as plsc

N = 10000
E = 320000
D = 128
G = 128
C = 10
MAX_LV = 3

NPAD = 10240
NC, NS = 2, 16
NW = NC * NS
BLK = 128
NBLK0 = 160
CH = 40
EPAD = NS * NBLK0 * BLK
ROWS_PER_TILE = NPAD // NS
ZROWS = 16

_mesh = plsc.VectorSubcoreMesh(core_axis_name="c", subcore_axis_name="s")


@functools.partial(
    pl.kernel,
    out_type=jax.ShapeDtypeStruct((NPAD, D), jnp.float32),
    mesh=_mesh,
    scratch_types=[
        pltpu.VMEM((CH, BLK), jnp.int32),
        pltpu.VMEM((CH, BLK), jnp.int32),
        [pltpu.VMEM((BLK, D), jnp.float32) for _ in range(2)],
        pltpu.VMEM((ZROWS, D), jnp.float32),
        pltpu.VMEM_SHARED((NPAD, D), jnp.float32),
        [pltpu.SemaphoreType.DMA for _ in range(2)],
        [pltpu.SemaphoreType.DMA for _ in range(2)],
    ],
)
def _sc_spmm(cur_hbm, srcw_hbm, dstw_hbm, z_hbm, out_hbm,
             src_v, dst_v, rows, zb, pooled, gsem, ssem):
    c = lax.axis_index("c")
    s = lax.axis_index("s")
    NB = 2

    with jax.named_scope("zero_phase"):
        pltpu.sync_copy(z_hbm, zb)
        row0 = s * ROWS_PER_TILE

        def _zero(k, carry):
            pltpu.sync_copy(zb, pooled.at[pl.ds(row0 + k * ZROWS, ZROWS)])
            return carry
        lax.fori_loop(0, ROWS_PER_TILE // ZROWS, _zero, 0)
        plsc.subcore_barrier()

    nchunks = jnp.where(c == 0, NBLK0 // CH, 0)

    def _chunk(ch, carry0):
        pltpu.sync_copy(srcw_hbm.at[s, pl.ds(ch * CH, CH)], src_v)
        pltpu.sync_copy(dstw_hbm.at[s, pl.ds(ch * CH, CH)], dst_v)
        for b in range(NB):
            pltpu.async_copy(cur_hbm.at[src_v.at[b]], rows[b], gsem[b])

        def _body(i, carry):
            j0 = NB * i
            for b in range(NB):
                j = j0 + b
                pltpu.make_async_copy(
                    cur_hbm.at[src_v.at[j]], rows[b], gsem[b]).wait()
                pltpu.async_copy(
                    rows[b], pooled.at[dst_v.at[j]], ssem[b], add=True)
            for b in range(NB):
                j = j0 + b + NB

                @pl.when(j < CH)
                def _():
                    pltpu.make_async_copy(
                        rows[b], pooled.at[dst_v.at[j0 + b]], ssem[b]).wait()
                    pltpu.async_copy(cur_hbm.at[src_v.at[j]], rows[b], gsem[b])
            return carry
        lax.fori_loop(0, CH // NB, _body, 0)
        for b in range(NB):
            pltpu.make_async_copy(
                rows[b], pooled.at[dst_v.at[CH - NB + b]], ssem[b]).wait()
        return carry0
    with jax.named_scope("edge_phase"):
        lax.fori_loop(0, nchunks, _chunk, 0)

    with jax.named_scope("writeout_phase"):
        plsc.subcore_barrier()

        @pl.when(c == 0)
        def _():
            pltpu.sync_copy(pooled.at[pl.ds(row0, ROWS_PER_TILE)],
                            out_hbm.at[pl.ds(row0, ROWS_PER_TILE)])


_RB = 1280


def _tc1_body(x_ref, w_ref, b_ref, im_ref, cur_ref):
    im = jnp.dot(x_ref[...], w_ref[...],
                 preferred_element_type=jnp.float32) + b_ref[...]
    im_ref[...] = im
    cur_ref[...] = jnp.maximum(im, 0.0)


def _tc1(x, w, b):
    return pl.pallas_call(
        _tc1_body,
        grid=(NPAD // _RB,),
        in_specs=[
            pl.BlockSpec((_RB, D), lambda i: (i, 0)),
            pl.BlockSpec((D, D), lambda i: (0, 0)),
            pl.BlockSpec((1, D), lambda i: (0, 0)),
        ],
        out_specs=[
            pl.BlockSpec((_RB, D), lambda i: (i, 0)),
            pl.BlockSpec((_RB, D), lambda i: (i, 0)),
        ],
        out_shape=[
            jax.ShapeDtypeStruct((NPAD, D), jnp.float32),
            jax.ShapeDtypeStruct((NPAD, D), jnp.float32),
        ],
    )(x, w, b)


def _tc2_body(p_ref, w_ref, b_ref, im_ref, cur_ref):
    nl = jnp.dot(p_ref[...], w_ref[...],
                 preferred_element_type=jnp.float32) + b_ref[...]
    cur_ref[...] = jnp.maximum(nl + im_ref[...], 0.0)


def _tc2(p, w, b, im):
    return pl.pallas_call(
        _tc2_body,
        grid=(NPAD // _RB,),
        in_specs=[
            pl.BlockSpec((_RB, D), lambda i: (i, 0)),
            pl.BlockSpec((D, D), lambda i: (0, 0)),
            pl.BlockSpec((1, D), lambda i: (0, 0)),
            pl.BlockSpec((_RB, D), lambda i: (i, 0)),
        ],
        out_specs=pl.BlockSpec((_RB, D), lambda i: (i, 0)),
        out_shape=jax.ShapeDtypeStruct((NPAD, D), jnp.float32),
    )(p, w, b, im)


def _tc3_body(cur_ref, gid_ref, wo_ref, bo_ref, wh1_ref, bh1_ref,
              wh2_ref, bh2_ref, out_ref, y_scr):
    i = pl.program_id(0)
    ra = jnp.maximum(
        jnp.dot(cur_ref[...], wo_ref[...],
                preferred_element_type=jnp.float32) + bo_ref[...], 0.0)
    g = gid_ref[0, 0, :]
    iota = lax.broadcasted_iota(jnp.int32, (G, _RB), 0)
    oh = (g[None, :] == iota).astype(jnp.float32)
    part = jnp.dot(oh, ra, preferred_element_type=jnp.float32)

    @pl.when(i == 0)
    def _():
        y_scr[...] = part

    @pl.when(i > 0)
    def _():
        y_scr[...] = y_scr[...] + part

    @pl.when(i == (NPAD // _RB) - 1)
    def _():
        ge = jnp.maximum(y_scr[...], 0.0)
        h1 = jnp.maximum(
            jnp.dot(ge, wh1_ref[...],
                    preferred_element_type=jnp.float32) + bh1_ref[...], 0.0)
        z = jnp.dot(h1, wh2_ref[...],
                    preferred_element_type=jnp.float32) + bh2_ref[...]
        m = jnp.max(z, axis=1, keepdims=True)
        lse = m + jnp.log(jnp.sum(jnp.exp(z - m), axis=1, keepdims=True))
        out_ref[...] = z - lse


def _tc3(cur, gid3, wo, bo, wh1, bh1, wh2, bh2):
    return pl.pallas_call(
        _tc3_body,
        grid=(NPAD // _RB,),
        in_specs=[
            pl.BlockSpec((_RB, D), lambda i: (i, 0)),
            pl.BlockSpec((1, 1, _RB), lambda i: (i, 0, 0)),
            pl.BlockSpec((D, D), lambda i: (0, 0)),
            pl.BlockSpec((1, D), lambda i: (0, 0)),
            pl.BlockSpec((D, D), lambda i: (0, 0)),
            pl.BlockSpec((1, D), lambda i: (0, 0)),
            pl.BlockSpec((D, C), lambda i: (0, 0)),
            pl.BlockSpec((1, C), lambda i: (0, 0)),
        ],
        out_specs=pl.BlockSpec((G, C), lambda i: (0, 0)),
        out_shape=jax.ShapeDtypeStruct((G, C), jnp.float32),
        scratch_shapes=[pltpu.VMEM((G, D), jnp.float32)],
    )(cur, gid3, wo, bo, wh1, bh1, wh2, bh2)


def kernel(node_feat, edge_index, graph_ids, W_n2l, b_n2l, W_conv, b_conv,
           W_out, b_out, W_h1, b_h1, W_h2, b_h2):
    f32 = jnp.float32
    nf = jnp.concatenate(
        [node_feat, jnp.zeros((NPAD - N, D), f32)], axis=0)
    src = jnp.concatenate(
        [edge_index[0], jnp.full((EPAD - E,), N, jnp.int32)])
    dst = jnp.concatenate(
        [edge_index[1], jnp.full((EPAD - E,), N, jnp.int32)])

    srcw = src.reshape(NS, NBLK0, BLK)
    dstw = dst.reshape(NS, NBLK0, BLK)
    zblk = jnp.zeros((ZROWS, D), f32)
    gid3 = jnp.concatenate(
        [graph_ids, jnp.full((NPAD - N,), G, jnp.int32)]).reshape(
            NPAD // _RB, 1, _RB)

    b_n2l2 = b_n2l.reshape(1, D)
    b_conv2 = b_conv.reshape(1, D)
    b_out2 = b_out.reshape(1, D)
    b_h12 = b_h1.reshape(1, D)
    b_h22 = b_h2.reshape(1, C)

    im, cur = _tc1(nf, W_n2l, b_n2l2)
    for _ in range(MAX_LV):
        pooled = _sc_spmm(cur, srcw, dstw, zblk)
        cur = _tc2(pooled, W_conv, b_conv2, im)
    return _tc3(cur, gid3, W_out, b_out2, W_h1, b_h12, W_h2, b_h22)


# speedup vs baseline: 3.9261x; 3.9261x over previous
import functools

import jax
import jax.numpy as jnp
from jax import lax
from jax.experimental import pallas as pl
from jax.experimental.pallas import tpu as pltpu
from jax.experimental.pallas import tpu_sc as plsc

N = 10000
E = 320000
D = 128
G = 128
C = 10
MAX_LV = 3

NPAD = 10240
NC, NS = 2, 16
NW = NC * NS
BLK = 128
NBLK = 80
CH = 40
EPAD = NW * NBLK * BLK
ROWS_PER_TILE = NPAD // NS
ZROWS = 16

_mesh = plsc.VectorSubcoreMesh(core_axis_name="c", subcore_axis_name="s")


@functools.partial(
    pl.kernel,
    out_type=jax.ShapeDtypeStruct((NC, NPAD, D), jnp.float32),
    mesh=_mesh,
    scratch_types=[
        pltpu.VMEM((CH, BLK), jnp.int32),
        pltpu.VMEM((CH, BLK), jnp.int32),
        [pltpu.VMEM((BLK, D), jnp.float32) for _ in range(2)],
        pltpu.VMEM((ZROWS, D), jnp.float32),
        pltpu.VMEM_SHARED((NPAD, D), jnp.float32),
        [pltpu.SemaphoreType.DMA for _ in range(2)],
        [pltpu.SemaphoreType.DMA for _ in range(2)],
    ],
)
def _sc_spmm(cur_hbm, srcw_hbm, dstw_hbm, z_hbm, out_hbm,
             src_v, dst_v, rows, zb, pooled, gsem, ssem):
    c = lax.axis_index("c")
    s = lax.axis_index("s")
    wid = s * NC + c
    NB = 2

    with jax.named_scope("zero_phase"):
        pltpu.sync_copy(z_hbm, zb)
        row0 = s * ROWS_PER_TILE

        def _zero(k, carry):
            pltpu.sync_copy(zb, pooled.at[pl.ds(row0 + k * ZROWS, ZROWS)])
            return carry
        lax.fori_loop(0, ROWS_PER_TILE // ZROWS, _zero, 0)
        plsc.subcore_barrier()

    nchunks = NBLK // CH

    def _chunk(ch, carry0):
        pltpu.sync_copy(srcw_hbm.at[wid, pl.ds(ch * CH, CH)], src_v)
        pltpu.sync_copy(dstw_hbm.at[wid, pl.ds(ch * CH, CH)], dst_v)
        for b in range(NB):
            pltpu.async_copy(cur_hbm.at[src_v.at[b]], rows[b], gsem[b])

        def _body(i, carry):
            j0 = NB * i
            for b in range(NB):
                j = j0 + b
                pltpu.make_async_copy(
                    cur_hbm.at[src_v.at[j]], rows[b], gsem[b]).wait()
                pltpu.async_copy(
                    rows[b], pooled.at[dst_v.at[j]], ssem[b], add=True)
            for b in range(NB):
                j = j0 + b + NB

                @pl.when(j < CH)
                def _():
                    pltpu.make_async_copy(
                        rows[b], pooled.at[dst_v.at[j0 + b]], ssem[b]).wait()
                    pltpu.async_copy(cur_hbm.at[src_v.at[j]], rows[b], gsem[b])
            return carry
        lax.fori_loop(0, CH // NB, _body, 0)
        for b in range(NB):
            pltpu.make_async_copy(
                rows[b], pooled.at[dst_v.at[CH - NB + b]], ssem[b]).wait()
        return carry0
    with jax.named_scope("edge_phase"):
        lax.fori_loop(0, nchunks, _chunk, 0)

    with jax.named_scope("writeout_phase"):
        plsc.subcore_barrier()
        pltpu.sync_copy(pooled.at[pl.ds(row0, ROWS_PER_TILE)],
                        out_hbm.at[c, pl.ds(row0, ROWS_PER_TILE)])


_RB = 1280


def _tc1_body(x_ref, w_ref, b_ref, im_ref, cur_ref):
    im = jnp.dot(x_ref[...], w_ref[...],
                 preferred_element_type=jnp.float32) + b_ref[...]
    im_ref[...] = im
    cur_ref[...] = jnp.maximum(im, 0.0)


def _tc1(x, w, b):
    return pl.pallas_call(
        _tc1_body,
        grid=(NPAD // _RB,),
        in_specs=[
            pl.BlockSpec((_RB, D), lambda i: (i, 0)),
            pl.BlockSpec((D, D), lambda i: (0, 0)),
            pl.BlockSpec((1, D), lambda i: (0, 0)),
        ],
        out_specs=[
            pl.BlockSpec((_RB, D), lambda i: (i, 0)),
            pl.BlockSpec((_RB, D), lambda i: (i, 0)),
        ],
        out_shape=[
            jax.ShapeDtypeStruct((NPAD, D), jnp.float32),
            jax.ShapeDtypeStruct((NPAD, D), jnp.float32),
        ],
    )(x, w, b)


def _tc2_body(p0_ref, p1_ref, w_ref, b_ref, im_ref, cur_ref):
    pooled = p0_ref[...] + p1_ref[...]
    nl = jnp.dot(pooled, w_ref[...],
                 preferred_element_type=jnp.float32) + b_ref[...]
    cur_ref[...] = jnp.maximum(nl + im_ref[...], 0.0)


def _tc2(p0, p1, w, b, im):
    return pl.pallas_call(
        _tc2_body,
        grid=(NPAD // _RB,),
        in_specs=[
            pl.BlockSpec((_RB, D), lambda i: (i, 0)),
            pl.BlockSpec((_RB, D), lambda i: (i, 0)),
            pl.BlockSpec((D, D), lambda i: (0, 0)),
            pl.BlockSpec((1, D), lambda i: (0, 0)),
            pl.BlockSpec((_RB, D), lambda i: (i, 0)),
        ],
        out_specs=pl.BlockSpec((_RB, D), lambda i: (i, 0)),
        out_shape=jax.ShapeDtypeStruct((NPAD, D), jnp.float32),
    )(p0, p1, w, b, im)


def _tc3_body(cur_ref, gid_ref, wo_ref, bo_ref, wh1_ref, bh1_ref,
              wh2_ref, bh2_ref, out_ref, y_scr):
    i = pl.program_id(0)
    ra = jnp.maximum(
        jnp.dot(cur_ref[...], wo_ref[...],
                preferred_element_type=jnp.float32) + bo_ref[...], 0.0)
    g = gid_ref[0, 0, :]
    iota = lax.broadcasted_iota(jnp.int32, (G, _RB), 0)
    oh = (g[None, :] == iota).astype(jnp.float32)
    part = jnp.dot(oh, ra, preferred_element_type=jnp.float32)

    @pl.when(i == 0)
    def _():
        y_scr[...] = part

    @pl.when(i > 0)
    def _():
        y_scr[...] = y_scr[...] + part

    @pl.when(i == (NPAD // _RB) - 1)
    def _():
        ge = jnp.maximum(y_scr[...], 0.0)
        h1 = jnp.maximum(
            jnp.dot(ge, wh1_ref[...],
                    preferred_element_type=jnp.float32) + bh1_ref[...], 0.0)
        z = jnp.dot(h1, wh2_ref[...],
                    preferred_element_type=jnp.float32) + bh2_ref[...]
        m = jnp.max(z, axis=1, keepdims=True)
        lse = m + jnp.log(jnp.sum(jnp.exp(z - m), axis=1, keepdims=True))
        out_ref[...] = z - lse


def _tc3(cur, gid3, wo, bo, wh1, bh1, wh2, bh2):
    return pl.pallas_call(
        _tc3_body,
        grid=(NPAD // _RB,),
        in_specs=[
            pl.BlockSpec((_RB, D), lambda i: (i, 0)),
            pl.BlockSpec((1, 1, _RB), lambda i: (i, 0, 0)),
            pl.BlockSpec((D, D), lambda i: (0, 0)),
            pl.BlockSpec((1, D), lambda i: (0, 0)),
            pl.BlockSpec((D, D), lambda i: (0, 0)),
            pl.BlockSpec((1, D), lambda i: (0, 0)),
            pl.BlockSpec((D, C), lambda i: (0, 0)),
            pl.BlockSpec((1, C), lambda i: (0, 0)),
        ],
        out_specs=pl.BlockSpec((G, C), lambda i: (0, 0)),
        out_shape=jax.ShapeDtypeStruct((G, C), jnp.float32),
        scratch_shapes=[pltpu.VMEM((G, D), jnp.float32)],
    )(cur, gid3, wo, bo, wh1, bh1, wh2, bh2)


def kernel(node_feat, edge_index, graph_ids, W_n2l, b_n2l, W_conv, b_conv,
           W_out, b_out, W_h1, b_h1, W_h2, b_h2):
    f32 = jnp.float32
    nf = jnp.concatenate(
        [node_feat, jnp.zeros((NPAD - N, D), f32)], axis=0)
    pad_rows = N + (jnp.arange(EPAD - E, dtype=jnp.int32) % (NPAD - N))
    src = jnp.concatenate([edge_index[0], pad_rows])
    dst = jnp.concatenate([edge_index[1], pad_rows])

    srcw = src.reshape(NW, NBLK, BLK)
    dstw = dst.reshape(NW, NBLK, BLK)
    zblk = jnp.zeros((ZROWS, D), f32)
    gid3 = jnp.concatenate(
        [graph_ids, jnp.full((NPAD - N,), G, jnp.int32)]).reshape(
            NPAD // _RB, 1, _RB)

    b_n2l2 = b_n2l.reshape(1, D)
    b_conv2 = b_conv.reshape(1, D)
    b_out2 = b_out.reshape(1, D)
    b_h12 = b_h1.reshape(1, D)
    b_h22 = b_h2.reshape(1, C)

    im, cur = _tc1(nf, W_n2l, b_n2l2)
    for _ in range(MAX_LV):
        parts = _sc_spmm(cur, srcw, dstw, zblk)
        cur = _tc2(parts[0], parts[1], W_conv, b_conv2, im)
    return _tc3(cur, gid3, W_out, b_out2, W_h1, b_h12, W_h2, b_h22)


# speedup vs baseline: 4.0847x; 1.0404x over previous
import functools

import jax
import jax.numpy as jnp
from jax import lax
from jax.experimental import pallas as pl
from jax.experimental.pallas import tpu as pltpu
from jax.experimental.pallas import tpu_sc as plsc

N = 10000
E = 320000
D = 128
G = 128
C = 10
MAX_LV = 3

NPAD = 10240
NC, NS = 2, 16
NW = NC * NS
BLK = 128
NBLK = 80
CH = 40
EPAD = NW * NBLK * BLK
ROWS_PER_TILE = NPAD // NS
ZROWS = 16

_mesh = plsc.VectorSubcoreMesh(core_axis_name="c", subcore_axis_name="s")


@functools.partial(
    pl.kernel,
    out_type=jax.ShapeDtypeStruct((NC, NPAD, D), jnp.float32),
    mesh=_mesh,
    scratch_types=[
        pltpu.VMEM((CH, BLK), jnp.int32),
        pltpu.VMEM((CH, BLK), jnp.int32),
        [pltpu.VMEM((BLK, D), jnp.float32) for _ in range(2)],
        pltpu.VMEM((ZROWS, D), jnp.float32),
        pltpu.VMEM_SHARED((NPAD, D), jnp.float32),
        [pltpu.SemaphoreType.DMA for _ in range(2)],
        [pltpu.SemaphoreType.DMA for _ in range(2)],
    ],
)
def _sc_spmm(cur_hbm, srcw_hbm, dstw_hbm, z_hbm, out_hbm,
             src_v, dst_v, rows, zb, pooled, gsem, ssem):
    c = lax.axis_index("c")
    s = lax.axis_index("s")
    wid = s * NC + c
    NB = 2

    with jax.named_scope("zero_phase"):
        pltpu.sync_copy(z_hbm, zb)
        row0 = s * ROWS_PER_TILE

        def _zero(k, carry):
            pltpu.sync_copy(zb, pooled.at[pl.ds(row0 + k * ZROWS, ZROWS)])
            return carry
        lax.fori_loop(0, ROWS_PER_TILE // ZROWS, _zero, 0)
        plsc.subcore_barrier()

    nchunks = NBLK // CH

    def _chunk(ch, carry0):
        pltpu.sync_copy(srcw_hbm.at[wid, pl.ds(ch * CH, CH)], src_v)
        pltpu.sync_copy(dstw_hbm.at[wid, pl.ds(ch * CH, CH)], dst_v)
        for b in range(NB):
            pltpu.async_copy(cur_hbm.at[src_v.at[b]], rows[b], gsem[b])

        def _body(i, carry):
            j0 = NB * i
            for b in range(NB):
                j = j0 + b
                pltpu.make_async_copy(
                    cur_hbm.at[src_v.at[j]], rows[b], gsem[b]).wait()
                pltpu.async_copy(
                    rows[b], pooled.at[dst_v.at[j]], ssem[b], add=True)
            for b in range(NB):
                j = j0 + b + NB

                @pl.when(j < CH)
                def _():
                    pltpu.make_async_copy(
                        rows[b], pooled.at[dst_v.at[j0 + b]], ssem[b]).wait()
                    pltpu.async_copy(cur_hbm.at[src_v.at[j]], rows[b], gsem[b])
            return carry
        lax.fori_loop(0, CH // NB, _body, 0)
        for b in range(NB):
            pltpu.make_async_copy(
                rows[b], pooled.at[dst_v.at[CH - NB + b]], ssem[b]).wait()
        return carry0
    with jax.named_scope("edge_phase"):
        lax.fori_loop(0, nchunks, _chunk, 0)

    with jax.named_scope("writeout_phase"):
        plsc.subcore_barrier()
        pltpu.sync_copy(pooled.at[pl.ds(row0, ROWS_PER_TILE)],
                        out_hbm.at[c, pl.ds(row0, ROWS_PER_TILE)])


_RB = 1280


def _tc1_body(x_ref, w_ref, b_ref, im_ref, cur_ref):
    im = jnp.dot(x_ref[...], w_ref[...],
                 preferred_element_type=jnp.float32) + b_ref[...]
    im_ref[...] = im
    cur_ref[...] = jnp.maximum(im, 0.0)


def _tc1(x, w, b):
    return pl.pallas_call(
        _tc1_body,
        grid=(NPAD // _RB,),
        in_specs=[
            pl.BlockSpec((_RB, D), lambda i: (i, 0)),
            pl.BlockSpec((D, D), lambda i: (0, 0)),
            pl.BlockSpec((1, D), lambda i: (0, 0)),
        ],
        out_specs=[
            pl.BlockSpec((_RB, D), lambda i: (i, 0)),
            pl.BlockSpec((_RB, D), lambda i: (i, 0)),
        ],
        out_shape=[
            jax.ShapeDtypeStruct((NPAD, D), jnp.float32),
            jax.ShapeDtypeStruct((NPAD, D), jnp.float32),
        ],
    )(x, w, b)


def _tc2_body(p0_ref, p1_ref, w_ref, b_ref, im_ref, cur_ref):
    pooled = p0_ref[0] + p1_ref[0]
    nl = jnp.dot(pooled, w_ref[...],
                 preferred_element_type=jnp.float32) + b_ref[...]
    cur_ref[...] = jnp.maximum(nl + im_ref[...], 0.0)


def _tc2(parts, w, b, im):
    return pl.pallas_call(
        _tc2_body,
        grid=(NPAD // _RB,),
        in_specs=[
            pl.BlockSpec((1, _RB, D), lambda i: (0, i, 0)),
            pl.BlockSpec((1, _RB, D), lambda i: (1, i, 0)),
            pl.BlockSpec((D, D), lambda i: (0, 0)),
            pl.BlockSpec((1, D), lambda i: (0, 0)),
            pl.BlockSpec((_RB, D), lambda i: (i, 0)),
        ],
        out_specs=pl.BlockSpec((_RB, D), lambda i: (i, 0)),
        out_shape=jax.ShapeDtypeStruct((NPAD, D), jnp.float32),
    )(parts, parts, w, b, im)


def _tc3_body(cur_ref, gid_ref, wo_ref, bo_ref, wh1_ref, bh1_ref,
              wh2_ref, bh2_ref, out_ref, y_scr):
    i = pl.program_id(0)
    ra = jnp.maximum(
        jnp.dot(cur_ref[...], wo_ref[...],
                preferred_element_type=jnp.float32) + bo_ref[...], 0.0)
    g = gid_ref[0, 0, :]
    iota = lax.broadcasted_iota(jnp.int32, (G, _RB), 0)
    oh = (g[None, :] == iota).astype(jnp.float32)
    part = jnp.dot(oh, ra, preferred_element_type=jnp.float32)

    @pl.when(i == 0)
    def _():
        y_scr[...] = part

    @pl.when(i > 0)
    def _():
        y_scr[...] = y_scr[...] + part

    @pl.when(i == (NPAD // _RB) - 1)
    def _():
        ge = jnp.maximum(y_scr[...], 0.0)
        h1 = jnp.maximum(
            jnp.dot(ge, wh1_ref[...],
                    preferred_element_type=jnp.float32) + bh1_ref[...], 0.0)
        z = jnp.dot(h1, wh2_ref[...],
                    preferred_element_type=jnp.float32) + bh2_ref[...]
        m = jnp.max(z, axis=1, keepdims=True)
        lse = m + jnp.log(jnp.sum(jnp.exp(z - m), axis=1, keepdims=True))
        out_ref[...] = z - lse


def _tc3(cur, gid3, wo, bo, wh1, bh1, wh2, bh2):
    return pl.pallas_call(
        _tc3_body,
        grid=(NPAD // _RB,),
        in_specs=[
            pl.BlockSpec((_RB, D), lambda i: (i, 0)),
            pl.BlockSpec((1, 1, _RB), lambda i: (i, 0, 0)),
            pl.BlockSpec((D, D), lambda i: (0, 0)),
            pl.BlockSpec((1, D), lambda i: (0, 0)),
            pl.BlockSpec((D, D), lambda i: (0, 0)),
            pl.BlockSpec((1, D), lambda i: (0, 0)),
            pl.BlockSpec((D, C), lambda i: (0, 0)),
            pl.BlockSpec((1, C), lambda i: (0, 0)),
        ],
        out_specs=pl.BlockSpec((G, C), lambda i: (0, 0)),
        out_shape=jax.ShapeDtypeStruct((G, C), jnp.float32),
        scratch_shapes=[pltpu.VMEM((G, D), jnp.float32)],
    )(cur, gid3, wo, bo, wh1, bh1, wh2, bh2)


def kernel(node_feat, edge_index, graph_ids, W_n2l, b_n2l, W_conv, b_conv,
           W_out, b_out, W_h1, b_h1, W_h2, b_h2):
    f32 = jnp.float32
    nf = jnp.concatenate(
        [node_feat, jnp.zeros((NPAD - N, D), f32)], axis=0)
    pad_rows = N + (jnp.arange(EPAD - E, dtype=jnp.int32) % (NPAD - N))
    src = jnp.concatenate([edge_index[0], pad_rows])
    dst = jnp.concatenate([edge_index[1], pad_rows])

    srcw = src.reshape(NW, NBLK, BLK)
    dstw = dst.reshape(NW, NBLK, BLK)
    zblk = jnp.zeros((ZROWS, D), f32)
    gid3 = jnp.concatenate(
        [graph_ids, jnp.full((NPAD - N,), G, jnp.int32)]).reshape(
            NPAD // _RB, 1, _RB)

    b_n2l2 = b_n2l.reshape(1, D)
    b_conv2 = b_conv.reshape(1, D)
    b_out2 = b_out.reshape(1, D)
    b_h12 = b_h1.reshape(1, D)
    b_h22 = b_h2.reshape(1, C)

    im, cur = _tc1(nf, W_n2l, b_n2l2)
    for _ in range(MAX_LV):
        parts = _sc_spmm(cur, srcw, dstw, zblk)
        cur = _tc2(parts, W_conv, b_conv2, im)
    return _tc3(cur, gid3, W_out, b_out2, W_h1, b_h12, W_h2, b_h22)


# speedup vs baseline: 4.7475x; 1.1623x over previous
import functools

import jax
import jax.numpy as jnp
from jax import lax
from jax.experimental import pallas as pl
from jax.experimental.pallas import tpu as pltpu
from jax.experimental.pallas import tpu_sc as plsc

N = 10000
E = 320000
D = 128
G = 128
C = 10
MAX_LV = 3

NPAD = 10240
NC, NS = 2, 16
NW = NC * NS
BLK = 64
NBLK = 160
CH = 40
EPAD = NW * NBLK * BLK
ROWS_PER_TILE = NPAD // NS
ZROWS = 16

_mesh = plsc.VectorSubcoreMesh(core_axis_name="c", subcore_axis_name="s")


@functools.partial(
    pl.kernel,
    out_type=jax.ShapeDtypeStruct((NC, NPAD, D), jnp.float32),
    mesh=_mesh,
    scratch_types=[
        pltpu.VMEM((CH, BLK), jnp.int32),
        pltpu.VMEM((CH, BLK), jnp.int32),
        [pltpu.VMEM((BLK, D), jnp.float32) for _ in range(4)],
        pltpu.VMEM((ZROWS, D), jnp.float32),
        pltpu.VMEM_SHARED((NPAD, D), jnp.float32),
        [pltpu.SemaphoreType.DMA for _ in range(4)],
        [pltpu.SemaphoreType.DMA for _ in range(4)],
    ],
)
def _sc_spmm(cur_hbm, srcw_hbm, dstw_hbm, z_hbm, out_hbm,
             src_v, dst_v, rows, zb, pooled, gsem, ssem):
    c = lax.axis_index("c")
    s = lax.axis_index("s")
    wid = s * NC + c
    NB = 4

    with jax.named_scope("zero_phase"):
        pltpu.sync_copy(z_hbm, zb)
        row0 = s * ROWS_PER_TILE

        def _zero(k, carry):
            pltpu.sync_copy(zb, pooled.at[pl.ds(row0 + k * ZROWS, ZROWS)])
            return carry
        lax.fori_loop(0, ROWS_PER_TILE // ZROWS, _zero, 0)
        plsc.subcore_barrier()

    nchunks = NBLK // CH

    def _chunk(ch, carry0):
        pltpu.sync_copy(srcw_hbm.at[wid, pl.ds(ch * CH, CH)], src_v)
        pltpu.sync_copy(dstw_hbm.at[wid, pl.ds(ch * CH, CH)], dst_v)
        for b in range(NB):
            pltpu.async_copy(cur_hbm.at[src_v.at[b]], rows[b], gsem[b])

        def _body(i, carry):
            j0 = NB * i
            for b in range(NB):
                j = j0 + b
                pltpu.make_async_copy(
                    cur_hbm.at[src_v.at[j]], rows[b], gsem[b]).wait()
                pltpu.async_copy(
                    rows[b], pooled.at[dst_v.at[j]], ssem[b], add=True)
            for b in range(NB):
                j = j0 + b + NB

                @pl.when(j < CH)
                def _():
                    pltpu.make_async_copy(
                        rows[b], pooled.at[dst_v.at[j0 + b]], ssem[b]).wait()
                    pltpu.async_copy(cur_hbm.at[src_v.at[j]], rows[b], gsem[b])
            return carry
        lax.fori_loop(0, CH // NB, _body, 0)
        for b in range(NB):
            pltpu.make_async_copy(
                rows[b], pooled.at[dst_v.at[CH - NB + b]], ssem[b]).wait()
        return carry0
    with jax.named_scope("edge_phase"):
        lax.fori_loop(0, nchunks, _chunk, 0)

    with jax.named_scope("writeout_phase"):
        plsc.subcore_barrier()
        pltpu.sync_copy(pooled.at[pl.ds(row0, ROWS_PER_TILE)],
                        out_hbm.at[c, pl.ds(row0, ROWS_PER_TILE)])


_RB = 1280


def _tc1_body(x_ref, w_ref, b_ref, im_ref, cur_ref):
    im = jnp.dot(x_ref[...], w_ref[...],
                 preferred_element_type=jnp.float32) + b_ref[...]
    im_ref[...] = im
    cur_ref[...] = jnp.maximum(im, 0.0)


def _tc1(x, w, b):
    return pl.pallas_call(
        _tc1_body,
        grid=(NPAD // _RB,),
        in_specs=[
            pl.BlockSpec((_RB, D), lambda i: (i, 0)),
            pl.BlockSpec((D, D), lambda i: (0, 0)),
            pl.BlockSpec((1, D), lambda i: (0, 0)),
        ],
        out_specs=[
            pl.BlockSpec((_RB, D), lambda i: (i, 0)),
            pl.BlockSpec((_RB, D), lambda i: (i, 0)),
        ],
        out_shape=[
            jax.ShapeDtypeStruct((NPAD, D), jnp.float32),
            jax.ShapeDtypeStruct((NPAD, D), jnp.float32),
        ],
    )(x, w, b)


def _tc2_body(p0_ref, p1_ref, w_ref, b_ref, im_ref, cur_ref):
    pooled = p0_ref[0] + p1_ref[0]
    nl = jnp.dot(pooled, w_ref[...],
                 preferred_element_type=jnp.float32) + b_ref[...]
    cur_ref[...] = jnp.maximum(nl + im_ref[...], 0.0)


def _tc2(parts, w, b, im):
    return pl.pallas_call(
        _tc2_body,
        grid=(NPAD // _RB,),
        in_specs=[
            pl.BlockSpec((1, _RB, D), lambda i: (0, i, 0)),
            pl.BlockSpec((1, _RB, D), lambda i: (1, i, 0)),
            pl.BlockSpec((D, D), lambda i: (0, 0)),
            pl.BlockSpec((1, D), lambda i: (0, 0)),
            pl.BlockSpec((_RB, D), lambda i: (i, 0)),
        ],
        out_specs=pl.BlockSpec((_RB, D), lambda i: (i, 0)),
        out_shape=jax.ShapeDtypeStruct((NPAD, D), jnp.float32),
    )(parts, parts, w, b, im)


def _tc3_body(cur_ref, gid_ref, wo_ref, bo_ref, wh1_ref, bh1_ref,
              wh2_ref, bh2_ref, out_ref, y_scr):
    i = pl.program_id(0)
    ra = jnp.maximum(
        jnp.dot(cur_ref[...], wo_ref[...],
                preferred_element_type=jnp.float32) + bo_ref[...], 0.0)
    g = gid_ref[0, 0, :]
    iota = lax.broadcasted_iota(jnp.int32, (G, _RB), 0)
    oh = (g[None, :] == iota).astype(jnp.float32)
    part = jnp.dot(oh, ra, preferred_element_type=jnp.float32)

    @pl.when(i == 0)
    def _():
        y_scr[...] = part

    @pl.when(i > 0)
    def _():
        y_scr[...] = y_scr[...] + part

    @pl.when(i == (NPAD // _RB) - 1)
    def _():
        ge = jnp.maximum(y_scr[...], 0.0)
        h1 = jnp.maximum(
            jnp.dot(ge, wh1_ref[...],
                    preferred_element_type=jnp.float32) + bh1_ref[...], 0.0)
        z = jnp.dot(h1, wh2_ref[...],
                    preferred_element_type=jnp.float32) + bh2_ref[...]
        m = jnp.max(z, axis=1, keepdims=True)
        lse = m + jnp.log(jnp.sum(jnp.exp(z - m), axis=1, keepdims=True))
        out_ref[...] = z - lse


def _tc3(cur, gid3, wo, bo, wh1, bh1, wh2, bh2):
    return pl.pallas_call(
        _tc3_body,
        grid=(NPAD // _RB,),
        in_specs=[
            pl.BlockSpec((_RB, D), lambda i: (i, 0)),
            pl.BlockSpec((1, 1, _RB), lambda i: (i, 0, 0)),
            pl.BlockSpec((D, D), lambda i: (0, 0)),
            pl.BlockSpec((1, D), lambda i: (0, 0)),
            pl.BlockSpec((D, D), lambda i: (0, 0)),
            pl.BlockSpec((1, D), lambda i: (0, 0)),
            pl.BlockSpec((D, C), lambda i: (0, 0)),
            pl.BlockSpec((1, C), lambda i: (0, 0)),
        ],
        out_specs=pl.BlockSpec((G, C), lambda i: (0, 0)),
        out_shape=jax.ShapeDtypeStruct((G, C), jnp.float32),
        scratch_shapes=[pltpu.VMEM((G, D), jnp.float32)],
    )(cur, gid3, wo, bo, wh1, bh1, wh2, bh2)


def kernel(node_feat, edge_index, graph_ids, W_n2l, b_n2l, W_conv, b_conv,
           W_out, b_out, W_h1, b_h1, W_h2, b_h2):
    f32 = jnp.float32
    nf = jnp.concatenate(
        [node_feat, jnp.zeros((NPAD - N, D), f32)], axis=0)
    pad_rows = N + (jnp.arange(EPAD - E, dtype=jnp.int32) % (NPAD - N))
    src = jnp.concatenate([edge_index[0], pad_rows])
    dst = jnp.concatenate([edge_index[1], pad_rows])

    srcw = src.reshape(NW, NBLK, BLK)
    dstw = dst.reshape(NW, NBLK, BLK)
    zblk = jnp.zeros((ZROWS, D), f32)
    gid3 = jnp.concatenate(
        [graph_ids, jnp.full((NPAD - N,), G, jnp.int32)]).reshape(
            NPAD // _RB, 1, _RB)

    b_n2l2 = b_n2l.reshape(1, D)
    b_conv2 = b_conv.reshape(1, D)
    b_out2 = b_out.reshape(1, D)
    b_h12 = b_h1.reshape(1, D)
    b_h22 = b_h2.reshape(1, C)

    im, cur = _tc1(nf, W_n2l, b_n2l2)
    for _ in range(MAX_LV):
        parts = _sc_spmm(cur, srcw, dstw, zblk)
        cur = _tc2(parts, W_conv, b_conv2, im)
    return _tc3(cur, gid3, W_out, b_out2, W_h1, b_h12, W_h2, b_h22)
